# baseline (speedup 1.0000x reference)
import functools

import jax
import jax.numpy as jnp
from jax import lax
from jax.experimental import pallas as pl
from jax.experimental.pallas import tpu as pltpu
from jax.experimental.pallas import tpu_sc as plsc

NC = 2
NS = 16
NW = NC * NS
L = 128
NBUF = 5


def kernel(X, W):
    B, H = X.shape
    V, D = W.shape
    N = B * H
    assert N % (L * NW) == 0
    rows = N // L
    rpw = rows // NW
    assert rpw % NBUF == 0
    ngrp = rpw // NBUF

    Xf = X.reshape(NW, rpw, L).astype(jnp.int32)

    mesh = plsc.VectorSubcoreMesh(core_axis_name="c", subcore_axis_name="s")

    @functools.partial(
        pl.kernel,
        out_type=jax.ShapeDtypeStruct((N, D), jnp.float32),
        mesh=mesh,
        scratch_types=[
            pltpu.VMEM((rpw, L), jnp.int32),
            [pltpu.VMEM((L, D), jnp.float32) for _ in range(NBUF)],
            [pltpu.SemaphoreType.DMA for _ in range(NBUF)],
            [pltpu.SemaphoreType.DMA for _ in range(NBUF)],
        ],
    )
    def emb(x_hbm, w_hbm, out_hbm, idx_v, bufs, gsems, osems):
        wid = lax.axis_index("s") * NC + lax.axis_index("c")
        row0 = wid * rpw
        pltpu.sync_copy(x_hbm.at[wid], idx_v)

        def start_gather(b, chunk):
            pltpu.make_async_copy(
                w_hbm.at[idx_v.at[chunk]], bufs[b], gsems[b]
            ).start()

        def start_out(b, chunk):
            pltpu.make_async_copy(
                bufs[b], out_hbm.at[pl.ds((row0 + chunk) * L, L)], osems[b]
            ).start()

        def wait_gather(b, chunk):
            pltpu.make_async_copy(
                w_hbm.at[idx_v.at[chunk]], bufs[b], gsems[b]
            ).wait()

        def wait_out(b, chunk):
            pltpu.make_async_copy(
                bufs[b], out_hbm.at[pl.ds((row0 + chunk) * L, L)], osems[b]
            ).wait()

        for b in range(NBUF):
            start_gather(b, b)

        def grp(g, carry):
            c0 = g * NBUF
            for b in range(NBUF):
                wait_gather(b, c0 + b)
                start_out(b, c0 + b)
            for b in range(NBUF):
                wait_out(b, c0 + b)
                start_gather(b, c0 + b + NBUF)
            return carry

        lax.fori_loop(0, ngrp - 1, grp, 0)

        c0 = (ngrp - 1) * NBUF
        for b in range(NBUF):
            wait_gather(b, c0 + b)
            start_out(b, c0 + b)
        for b in range(NBUF):
            wait_out(b, c0 + b)

    out = emb(Xf, W)
    return out.reshape(B, H, D)


# speedup vs baseline: 1.7154x; 1.7154x over previous
import functools

import jax
import jax.numpy as jnp
from jax import lax
from jax.experimental import pallas as pl
from jax.experimental.pallas import tpu as pltpu
from jax.experimental.pallas import tpu_sc as plsc

NC = 2
NS = 16
NW = NC * NS
KB = 8
NBUF = 2


def kernel(X, W):
    B, H = X.shape
    V, D = W.shape
    bpw = B // NW
    nch = bpw // KB
    ngrp = nch // NBUF
    assert bpw * NW == B and KB * nch == bpw and NBUF * ngrp == nch

    Xi = X.astype(jnp.int32)

    mesh = plsc.VectorSubcoreMesh(core_axis_name="c", subcore_axis_name="s")

    @functools.partial(
        pl.kernel,
        out_type=jax.ShapeDtypeStruct((B, H, D), jnp.float32),
        mesh=mesh,
        scratch_types=[
            pltpu.VMEM((bpw, H), jnp.int32),
            [pltpu.VMEM((KB, H, D), jnp.float32) for _ in range(NBUF)],
            [pltpu.SemaphoreType.DMA for _ in range(NBUF)],
            [pltpu.SemaphoreType.DMA for _ in range(NBUF)],
        ],
    )
    def emb(x_hbm, w_hbm, out_hbm, idx_v, bufs, gsems, osems):
        wid = lax.axis_index("s") * NC + lax.axis_index("c")
        b0 = wid * bpw
        pltpu.sync_copy(x_hbm.at[pl.ds(b0, bpw)], idx_v)

        def fire(b, c):
            for jj in range(KB):
                pltpu.make_async_copy(
                    w_hbm.at[idx_v.at[c * KB + jj]], bufs[b].at[jj], gsems[b]
                ).start()

        def drain_and_out(b, c):
            for jj in range(KB):
                pltpu.make_async_copy(
                    w_hbm.at[idx_v.at[c * KB + jj]], bufs[b].at[jj], gsems[b]
                ).wait()
            pltpu.make_async_copy(
                bufs[b], out_hbm.at[pl.ds(b0 + c * KB, KB)], osems[b]
            ).start()

        def wait_out(b, c):
            pltpu.make_async_copy(
                bufs[b], out_hbm.at[pl.ds(b0 + c * KB, KB)], osems[b]
            ).wait()

        for b in range(NBUF):
            fire(b, b)

        def grp(g, carry):
            c0 = g * NBUF
            for b in range(NBUF):
                drain_and_out(b, c0 + b)
            for b in range(NBUF):
                wait_out(b, c0 + b)
                fire(b, c0 + b + NBUF)
            return carry

        lax.fori_loop(0, ngrp - 1, grp, 0)

        c0 = (ngrp - 1) * NBUF
        for b in range(NBUF):
            drain_and_out(b, c0 + b)
        for b in range(NBUF):
            wait_out(b, c0 + b)

    return emb(Xi, W)


# speedup vs baseline: 3.1384x; 1.8295x over previous
import functools

import jax
import jax.numpy as jnp
from jax import lax
from jax.experimental import pallas as pl
from jax.experimental.pallas import tpu as pltpu
from jax.experimental.pallas import tpu_sc as plsc

NC = 2
NS = 16
NW = NC * NS
L = 128
NBUF = 5


def kernel(X, W):
    B, H = X.shape
    V, D = W.shape
    bpw = B // NW
    assert bpw * NW == B and bpw == L and H % NBUF == 0
    ngrp = H // NBUF

    Xt = X.T.astype(jnp.int32)

    mesh = plsc.VectorSubcoreMesh(core_axis_name="c", subcore_axis_name="s")

    @functools.partial(
        pl.kernel,
        out_type=jax.ShapeDtypeStruct((H, B, D), jnp.float32),
        mesh=mesh,
        scratch_types=[
            pltpu.VMEM((H, L), jnp.int32),
            [pltpu.VMEM((L, D), jnp.float32) for _ in range(NBUF)],
            [pltpu.SemaphoreType.DMA for _ in range(NBUF)],
            [pltpu.SemaphoreType.DMA for _ in range(NBUF)],
        ],
    )
    def emb(x_hbm, w_hbm, out_hbm, idx_v, bufs, gsems, osems):
        wid = lax.axis_index("s") * NC + lax.axis_index("c")
        b0 = wid * L
        pltpu.sync_copy(x_hbm.at[:, pl.ds(b0, L)], idx_v)

        def start_gather(b, h):
            pltpu.make_async_copy(
                w_hbm.at[idx_v.at[h]], bufs[b], gsems[b]
            ).start()

        def wait_gather(b, h):
            pltpu.make_async_copy(
                w_hbm.at[idx_v.at[h]], bufs[b], gsems[b]
            ).wait()

        def start_out(b, h):
            pltpu.make_async_copy(
                bufs[b], out_hbm.at[h, pl.ds(b0, L)], osems[b]
            ).start()

        def wait_out(b, h):
            pltpu.make_async_copy(
                bufs[b], out_hbm.at[h, pl.ds(b0, L)], osems[b]
            ).wait()

        for b in range(NBUF):
            start_gather(b, b)

        def grp(g, carry):
            h0 = g * NBUF
            for b in range(NBUF):
                wait_gather(b, h0 + b)
                start_out(b, h0 + b)
            for b in range(NBUF):
                wait_out(b, h0 + b)
                start_gather(b, h0 + b + NBUF)
            return carry

        lax.fori_loop(0, ngrp - 1, grp, 0)

        h0 = (ngrp - 1) * NBUF
        for b in range(NBUF):
            wait_gather(b, h0 + b)
            start_out(b, h0 + b)
        for b in range(NBUF):
            wait_out(b, h0 + b)

    out = emb(Xt, W)
    return jnp.transpose(out, (1, 0, 2))
